# baseline (speedup 1.0000x reference)
import functools

import jax
import jax.numpy as jnp
from jax import lax
from jax.experimental import pallas as pl
from jax.experimental.pallas import tpu as pltpu
from jax.experimental.pallas import tpu_sc as plsc

NBPS = 4
CLIP = 20.0
NC, NS = 2, 16
NW = NC * NS


def _lane_swap8(x):
    idx = lax.iota(jnp.int32, 16) ^ 8
    dnums = lax.GatherDimensionNumbers(offset_dims=(),
                                       collapsed_slice_dims=(0,),
                                       start_index_map=(0,))
    return lax.gather(x, idx[:, None], dnums, (1,),
                      mode=lax.GatherScatterMode.PROMISE_IN_BOUNDS)


def _build_sc(B, K, S):
    bpw = B // NW
    pi_rows = bpw * K * S // 128
    d_rows = bpw * K // 128
    out_rows = bpw * S * NBPS // 128
    d_base = (B * K * S) // 128
    mesh = plsc.VectorSubcoreMesh(core_axis_name="c", subcore_axis_name="s",
                                  num_cores=NC, num_subcores=NS)

    @functools.partial(
        pl.kernel,
        out_type=jax.ShapeDtypeStruct((B * S * NBPS // 128, 128), jnp.float32),
        mesh=mesh,
        scratch_types=[
            pltpu.VMEM((pi_rows, 128), jnp.int32),
            pltpu.VMEM((d_rows, 128), jnp.int32),
            pltpu.VMEM((out_rows, 128), jnp.float32),
        ],
        compiler_params=pltpu.CompilerParams(needs_layout_passes=False,
                                             use_tc_tiling_on_sc=False),
    )
    def llr_kernel(combo_hbm, out_hbm, pi_v, d_v, out_v):
        wid = lax.axis_index("s") * NC + lax.axis_index("c")
        pltpu.sync_copy(combo_hbm.at[pl.ds(wid * pi_rows, pi_rows)], pi_v)
        pltpu.sync_copy(combo_hbm.at[pl.ds(d_base + wid * d_rows, d_rows)],
                        d_v)

        iota = lax.iota(jnp.int32, 16)
        hi = iota >> 3
        lane_lt8 = iota < 8
        inf = jnp.full((16,), jnp.inf, jnp.float32)
        oidx = [(iota & 7) * NBPS + i for i in range(NBPS)]

        UNROLL = 8

        def row(b, carry):
            d_row = jnp.zeros((16,), jnp.int32) + (b >> 1)
            d_lane0 = (b & 1) * 64
            out_row = jnp.zeros((16,), jnp.int32) + (b >> 2)
            obase = (b & 3) * 32

            def jstep(jc, accs):
                a0, a1 = list(accs[0]), list(accs[1])
                pi_row = b * 4 + jc
                for u in range(UNROLL):
                    j = jc * UNROLL + u
                    v = pi_v[pi_row, pl.ds(16 * u, 16)]
                    dj = plsc.bitcast(
                        plsc.load_gather(d_v,
                                         [d_row, hi + (d_lane0 + 2 * j)]),
                        jnp.float32)
                    for i in range(NBPS):
                        m0 = (v & (8 >> i)) == 0
                        a0[i] = jnp.minimum(a0[i], jnp.where(m0, dj, inf))
                        a1[i] = jnp.minimum(a1[i], jnp.where(m0, inf, dj))
                return (tuple(a0), tuple(a1))

            a0, a1 = lax.fori_loop(0, K // 2 // UNROLL, jstep,
                                   ((inf,) * NBPS, (inf,) * NBPS))
            for i in range(NBPS):
                f0 = jnp.minimum(a0[i], _lane_swap8(a0[i]))
                f1 = jnp.minimum(a1[i], _lane_swap8(a1[i]))
                llr = jnp.clip((f0 - f1) * 0.5, -CLIP, CLIP)
                plsc.store_scatter(out_v, [out_row, oidx[i] + obase],
                                   llr, mask=lane_lt8)
            return carry

        lax.fori_loop(0, bpw, row, 0)
        pltpu.sync_copy(out_v, out_hbm.at[pl.ds(wid * out_rows, out_rows)])

    return llr_kernel


def kernel(y, r, dists, path_inds, path_syms):
    B, K, S = path_inds.shape
    pi = path_inds.reshape(B * K * S // 128, 128)
    dd = lax.bitcast_convert_type(dists, jnp.int32).reshape(B * K // 128, 128)
    combo = jnp.concatenate([pi, dd], axis=0)
    out = _build_sc(B, K, S)(combo)
    return out.reshape(B, S, NBPS)


# speedup vs baseline: 1.0654x; 1.0654x over previous
import functools

import jax
import jax.numpy as jnp
from jax import lax
from jax.experimental import pallas as pl
from jax.experimental.pallas import tpu as pltpu
from jax.experimental.pallas import tpu_sc as plsc

NBPS = 4
CLIP = 20.0
NC, NS = 2, 16
NW = NC * NS


def _lane_perm(x, idx):
    dnums = lax.GatherDimensionNumbers(offset_dims=(),
                                       collapsed_slice_dims=(0,),
                                       start_index_map=(0,))
    return lax.gather(x, idx[:, None], dnums, (1,),
                      mode=lax.GatherScatterMode.PROMISE_IN_BOUNDS)


def _build_sc(B, K, S):
    bpw = B // NW
    mesh = plsc.VectorSubcoreMesh(core_axis_name="c", subcore_axis_name="s",
                                  num_cores=NC, num_subcores=NS)

    @functools.partial(
        pl.kernel,
        out_type=jax.ShapeDtypeStruct((B, S, NBPS), jnp.float32),
        mesh=mesh,
        scratch_types=[
            pltpu.VMEM((bpw, K, S), jnp.int32),
            pltpu.VMEM((bpw, K), jnp.float32),
            pltpu.VMEM((bpw, S, NBPS), jnp.float32),
        ],
        compiler_params=pltpu.CompilerParams(needs_layout_passes=False,
                                             use_tc_tiling_on_sc=False),
    )
    def llr_kernel(pi_hbm, d_hbm, out_hbm, pi_v, d_v, out_v):
        wid = lax.axis_index("s") * NC + lax.axis_index("c")
        base = wid * bpw
        pltpu.sync_copy(pi_hbm.at[pl.ds(base, bpw)], pi_v)
        pltpu.sync_copy(d_hbm.at[pl.ds(base, bpw)], d_v)

        iota = lax.iota(jnp.int32, 16)
        hi = iota >> 3
        lane_s = iota & 7
        lane_lt8 = iota < 8
        swap8 = iota ^ 8
        perms = [hi + 2 * u for u in range(8)]
        inf = jnp.full((16,), jnp.inf, jnp.float32)
        splat_i = [jnp.full((16,), i, jnp.int32) for i in range(NBPS)]

        UNROLL = 8

        def row(b, carry):
            splat_b = jnp.zeros((16,), jnp.int32) + b

            def jstep(jc, accs):
                a0, a1 = list(accs[0]), list(accs[1])
                d16 = d_v[b, pl.ds(16 * jc, 16)]
                for u in range(UNROLL):
                    j = jc * UNROLL + u
                    ik = hi + 2 * j
                    v = plsc.load_gather(pi_v, [splat_b, ik, lane_s])
                    dj = _lane_perm(d16, perms[u])
                    for i in range(NBPS):
                        m0 = (v & (8 >> i)) == 0
                        a0[i] = jnp.minimum(a0[i], jnp.where(m0, dj, inf))
                        a1[i] = jnp.minimum(a1[i], jnp.where(m0, inf, dj))
                return (tuple(a0), tuple(a1))

            a0, a1 = lax.fori_loop(0, K // 2 // UNROLL, jstep,
                                   ((inf,) * NBPS, (inf,) * NBPS))
            for i in range(NBPS):
                f0 = jnp.minimum(a0[i], _lane_perm(a0[i], swap8))
                f1 = jnp.minimum(a1[i], _lane_perm(a1[i], swap8))
                llr = jnp.clip((f0 - f1) * 0.5, -CLIP, CLIP)
                plsc.store_scatter(out_v, [splat_b, lane_s, splat_i[i]],
                                   llr, mask=lane_lt8)
            return carry

        lax.fori_loop(0, bpw, row, 0)
        pltpu.sync_copy(out_v, out_hbm.at[pl.ds(base, bpw)])

    return llr_kernel


def kernel(y, r, dists, path_inds, path_syms):
    B, K, S = path_inds.shape
    return _build_sc(B, K, S)(path_inds, dists)


# speedup vs baseline: 1.1028x; 1.0352x over previous
import functools

import jax
import jax.numpy as jnp
from jax import lax
from jax.experimental import pallas as pl
from jax.experimental.pallas import tpu as pltpu
from jax.experimental.pallas import tpu_sc as plsc

NBPS = 4
CLIP = 20.0
NC, NS = 2, 16
NW = NC * NS


def _build(B, K, S):
    bpw = B // NW
    mesh = plsc.VectorSubcoreMesh(core_axis_name="c", subcore_axis_name="s",
                                  num_cores=NC, num_subcores=NS)

    @functools.partial(
        pl.kernel,
        out_type=jax.ShapeDtypeStruct((B, S, NBPS), jnp.float32),
        mesh=mesh,
        scratch_types=[
            pltpu.VMEM((bpw, K, S), jnp.int32),
            pltpu.VMEM((bpw, K), jnp.float32),
            pltpu.VMEM((bpw, S, NBPS), jnp.float32),
            pltpu.VMEM((24,), jnp.float32),
        ],
        compiler_params=pltpu.CompilerParams(needs_layout_passes=False,
                                             use_tc_tiling_on_sc=False),
    )
    def llr_kernel(pi_hbm, d_hbm, out_hbm, pi_v, d_v, out_v, fold_v):
        wid = lax.axis_index("s") * NC + lax.axis_index("c")
        base = wid * bpw
        pltpu.sync_copy(pi_hbm.at[pl.ds(base, bpw)], pi_v)
        pltpu.sync_copy(d_hbm.at[pl.ds(base, bpw)], d_v)

        iota = lax.iota(jnp.int32, 16)
        hi = iota >> 3
        lane_s = iota & 7
        lane_lt8 = iota < 8
        inf = jnp.full((16,), jnp.inf, jnp.float32)
        splat_i = [jnp.full((16,), i, jnp.int32) for i in range(NBPS)]

        UNROLL = 4

        def row(b, carry):
            splat_b = jnp.zeros((16,), jnp.int32) + b

            def jstep(jc, accs):
                a0, a1 = list(accs[0]), list(accs[1])
                for u in range(UNROLL):
                    j = jc * UNROLL + u
                    ik = hi + 2 * j
                    v = plsc.load_gather(pi_v, [splat_b, ik, lane_s])
                    dj = plsc.load_gather(d_v, [splat_b, ik])
                    for i in range(NBPS):
                        m0 = (v & (8 >> i)) == 0
                        a0[i] = jnp.minimum(a0[i], jnp.where(m0, dj, inf))
                        a1[i] = jnp.minimum(a1[i], jnp.where(m0, inf, dj))
                return (tuple(a0), tuple(a1))

            a0, a1 = lax.fori_loop(0, K // 2 // UNROLL, jstep,
                                   ((inf,) * NBPS, (inf,) * NBPS))
            for i in range(NBPS):
                fold_v[pl.ds(0, 16)] = a0[i]
                f0 = jnp.minimum(a0[i], fold_v[pl.ds(8, 16)])
                fold_v[pl.ds(0, 16)] = a1[i]
                f1 = jnp.minimum(a1[i], fold_v[pl.ds(8, 16)])
                llr = jnp.clip((f0 - f1) * 0.5, -CLIP, CLIP)
                plsc.store_scatter(out_v, [splat_b, lane_s, splat_i[i]],
                                   llr, mask=lane_lt8)
            return carry

        lax.fori_loop(0, bpw, row, 0)
        pltpu.sync_copy(out_v, out_hbm.at[pl.ds(base, bpw)])

    return llr_kernel


def kernel(y, r, dists, path_inds, path_syms):
    B, K, S = path_inds.shape
    return _build(B, K, S)(path_inds, dists)
